# Initial kernel scaffold; baseline (speedup 1.0000x reference)
import jax
import jax.numpy as jnp
from jax.experimental import pallas as pl


def kernel(x, edge_index, W1, b1, g1, be1, rm1, rv1, W2, b2, g2, be2, rm2, rv2, W3, b3):
    raise NotImplementedError("write your pallas kernel here")


# speedup vs baseline: 12.9197x; 12.9197x over previous
import functools

import jax
import jax.numpy as jnp
from jax import lax
from jax.experimental import pallas as pl
from jax.experimental.pallas import tpu as pltpu
from jax.experimental.pallas import tpu_sc as plsc

N = 10000
E = 320000
D = 128
H = 128
C = 40
BN_EPS = 1e-5

NC = 2
NS = 16
NW = NC * NS
NP = 10240
RPT = NP // NS
EPW = E // NW
CHUNK = 80
NCHUNK = EPW // CHUNK
ERODS = E // CHUNK
BLK = 640
GRID = NP // BLK

_mesh = plsc.VectorSubcoreMesh(core_axis_name="c", subcore_axis_name="s")


@functools.partial(
    pl.kernel,
    out_type=jax.ShapeDtypeStruct((NC * NP,), jnp.float32),
    mesh=_mesh,
    scratch_types=[
        pltpu.VMEM_SHARED((NP,), jnp.float32),
        pltpu.VMEM((CHUNK,), jnp.int32),
        pltpu.VMEM((CHUNK,), jnp.float32),
        pltpu.VMEM((RPT,), jnp.float32),
    ],
)
def _sc_degree(dst_hbm, out_hbm, deg_sh, dsti_v, ones_v, zeros_v):
    c = lax.axis_index("c")
    s = lax.axis_index("s")
    wid = c * NS + s

    def fill_zero(i, _):
        zeros_v[pl.ds(i * 16, 16)] = jnp.zeros((16,), jnp.float32)
        return _
    lax.fori_loop(0, RPT // 16, fill_zero, None)

    def fill_one(i, _):
        ones_v[pl.ds(i * 16, 16)] = jnp.full((16,), 1.0, jnp.float32)
        return _
    lax.fori_loop(0, CHUNK // 16, fill_one, None)

    pltpu.sync_copy(zeros_v, deg_sh.at[pl.ds(s * RPT, RPT)])
    plsc.subcore_barrier()

    def body(k, _):
        pltpu.sync_copy(dst_hbm.at[pl.ds(wid * EPW + k * CHUNK, CHUNK)],
                        dsti_v)
        pltpu.sync_copy(ones_v, deg_sh.at[dsti_v], add=True)
        return _
    lax.fori_loop(0, NCHUNK, body, None)

    plsc.subcore_barrier()
    pltpu.sync_copy(deg_sh.at[pl.ds(s * RPT, RPT)],
                    out_hbm.at[pl.ds(c * NP + s * RPT, RPT)])


@functools.partial(
    pl.kernel,
    out_type=jax.ShapeDtypeStruct((NC * NP, H), jnp.float32),
    mesh=_mesh,
    scratch_types=[
        pltpu.VMEM_SHARED((NP, H), jnp.float32),
        pltpu.VMEM((CHUNK,), jnp.int32),
        pltpu.VMEM((CHUNK,), jnp.int32),
        pltpu.VMEM((CHUNK, H), jnp.float32),
        pltpu.VMEM((64, H), jnp.float32),
        pltpu.SemaphoreType.DMA,
    ],
)
def _sc_aggregate(y_hbm, src_hbm, dst_hbm, out_hbm,
                  acc_sh, srci_v, dsti_v, rows_v, zrows_v, sem):
    c = lax.axis_index("c")
    s = lax.axis_index("s")
    wid = c * NS + s

    def fill_zero(i, _):
        for j in range(H // 16):
            zrows_v[i, pl.ds(j * 16, 16)] = jnp.zeros((16,), jnp.float32)
        return _
    lax.fori_loop(0, 64, fill_zero, None)

    def zero_acc(j, _):
        pltpu.sync_copy(zrows_v, acc_sh.at[pl.ds(s * RPT + j * 64, 64)])
        return _
    lax.fori_loop(0, RPT // 64, zero_acc, None)
    plsc.subcore_barrier()

    def body(k, _):
        base = wid * EPW + k * CHUNK
        pltpu.sync_copy(src_hbm.at[pl.ds(base, CHUNK)], srci_v)
        pltpu.sync_copy(dst_hbm.at[pl.ds(base, CHUNK)], dsti_v)
        pltpu.async_copy(y_hbm.at[srci_v], rows_v, sem).wait()
        pltpu.sync_copy(rows_v, acc_sh.at[dsti_v], add=True)
        return _
    lax.fori_loop(0, NCHUNK, body, None)

    plsc.subcore_barrier()
    pltpu.sync_copy(acc_sh.at[pl.ds(s * RPT, RPT)],
                    out_hbm.at[pl.ds(c * NP + s * RPT, RPT)])


def _tc1_body(degA, degB, x_ref, w_ref, y_ref, dinv_ref):
    deg = degA[...] + degB[...] + 1.0
    dinv = lax.rsqrt(deg)
    dinv_ref[...] = jnp.broadcast_to(dinv, (BLK, H))
    h = jnp.dot(x_ref[...], w_ref[...], preferred_element_type=jnp.float32)
    y_ref[...] = h * dinv


def _tc1(deg2, x_pad, W1):
    return pl.pallas_call(
        _tc1_body,
        grid=(GRID,),
        in_specs=[
            pl.BlockSpec((BLK, 1), lambda i: (i, 0)),
            pl.BlockSpec((BLK, 1), lambda i: (i + GRID, 0)),
            pl.BlockSpec((BLK, D), lambda i: (i, 0)),
            pl.BlockSpec((D, H), lambda i: (0, 0)),
        ],
        out_specs=[
            pl.BlockSpec((BLK, H), lambda i: (i, 0)),
            pl.BlockSpec((BLK, H), lambda i: (i, 0)),
        ],
        out_shape=[
            jax.ShapeDtypeStruct((NP, H), jnp.float32),
            jax.ShapeDtypeStruct((NP, H), jnp.float32),
        ],
        compiler_params=pltpu.CompilerParams(
            dimension_semantics=("arbitrary",)),
    )(deg2, deg2, x_pad, W1)


def _tc2_body(aggA, aggB, y_ref, dinv_ref, b_ref, g_ref, be_ref, rm_ref,
              rv_ref, w_ref, out_ref):
    dinv = dinv_ref[...]
    t = dinv * (aggA[...] + aggB[...] + y_ref[...]) + b_ref[...]
    t = jnp.maximum(t, 0.0)
    t = (t - rm_ref[...]) * lax.rsqrt(rv_ref[...] + BN_EPS) * g_ref[...] \
        + be_ref[...]
    t = jnp.maximum(t, 0.0)
    h = jnp.dot(t, w_ref[...], preferred_element_type=jnp.float32)
    out_ref[...] = h * dinv


def _tc2(aggs, y1, dinv_b, b1, g1, be1, rm1, rv1, W2):
    row = pl.BlockSpec((1, H), lambda i: (0, 0))
    return pl.pallas_call(
        _tc2_body,
        grid=(GRID,),
        in_specs=[
            pl.BlockSpec((BLK, H), lambda i: (i, 0)),
            pl.BlockSpec((BLK, H), lambda i: (i + GRID, 0)),
            pl.BlockSpec((BLK, H), lambda i: (i, 0)),
            pl.BlockSpec((BLK, H), lambda i: (i, 0)),
            row, row, row, row, row,
            pl.BlockSpec((H, H), lambda i: (0, 0)),
        ],
        out_specs=pl.BlockSpec((BLK, H), lambda i: (i, 0)),
        out_shape=jax.ShapeDtypeStruct((NP, H), jnp.float32),
        compiler_params=pltpu.CompilerParams(
            dimension_semantics=("arbitrary",)),
    )(aggs, aggs, y1, dinv_b, b1, g1, be1, rm1, rv1, W2)


def _tc3_body(aggA, aggB, y_ref, dinv_ref, b_ref, g_ref, be_ref, rm_ref,
              rv_ref, w_ref, b3_ref, logp_ref, var_ref, acc):
    i = pl.program_id(0)

    @pl.when(i == 0)
    def _():
        acc[0] = 0.0
        acc[1] = 0.0

    t = dinv_ref[...] * (aggA[...] + aggB[...] + y_ref[...]) + b_ref[...]
    t = jnp.maximum(t, 0.0)
    t = (t - rm_ref[...]) * lax.rsqrt(rv_ref[...] + BN_EPS) * g_ref[...] \
        + be_ref[...]
    t = jnp.maximum(t, 0.0)
    o = jnp.dot(t, w_ref[...], preferred_element_type=jnp.float32) + b3_ref[...]

    col = lax.broadcasted_iota(jnp.int32, (BLK, H), 1)
    colmask = col < C
    om = jnp.where(colmask, o, -1e30)
    m = jnp.max(om, axis=1, keepdims=True)
    ex = jnp.where(colmask, jnp.exp(o - m), 0.0)
    se = jnp.sum(ex, axis=1, keepdims=True)
    logp_ref[...] = o - m - jnp.log(se)

    rowg = lax.broadcasted_iota(jnp.int32, (BLK, H), 0) + i * BLK
    vmask = colmask & (rowg < N)
    ov = jnp.where(vmask, o, 0.0)
    acc[0] += jnp.sum(ov)
    acc[1] += jnp.sum(ov * ov)

    n = float(N * C)
    v = (acc[1] - acc[0] * acc[0] / n) / (n - 1.0)
    var_ref[...] = jnp.full((1, 1), v, jnp.float32)


def _tc3(aggs, y2, dinv_b, b2, g2, be2, rm2, rv2, W3p, b3p):
    row = pl.BlockSpec((1, H), lambda i: (0, 0))
    return pl.pallas_call(
        _tc3_body,
        grid=(GRID,),
        in_specs=[
            pl.BlockSpec((BLK, H), lambda i: (i, 0)),
            pl.BlockSpec((BLK, H), lambda i: (i + GRID, 0)),
            pl.BlockSpec((BLK, H), lambda i: (i, 0)),
            pl.BlockSpec((BLK, H), lambda i: (i, 0)),
            row, row, row, row, row,
            pl.BlockSpec((H, H), lambda i: (0, 0)),
            row,
        ],
        out_specs=[
            pl.BlockSpec((BLK, H), lambda i: (i, 0)),
            pl.BlockSpec((1, 1), lambda i: (0, 0)),
        ],
        out_shape=[
            jax.ShapeDtypeStruct((NP, H), jnp.float32),
            jax.ShapeDtypeStruct((1, 1), jnp.float32),
        ],
        scratch_shapes=[pltpu.SMEM((2,), jnp.float32)],
        compiler_params=pltpu.CompilerParams(
            dimension_semantics=("arbitrary",)),
    )(aggs, aggs, y2, dinv_b, b2, g2, be2, rm2, rv2, W3p, b3p)


def kernel(x, edge_index, W1, b1, g1, be1, rm1, rv1, W2, b2, g2, be2, rm2,
           rv2, W3, b3):
    src = edge_index[0]
    dst = edge_index[1]
    x_pad = jnp.pad(x, ((0, NP - N), (0, 0)))

    row = lambda v: v.reshape(1, H)
    W3p = jnp.pad(W3, ((0, 0), (0, H - C)))
    b3p = jnp.pad(b3, (0, H - C)).reshape(1, H)

    degs = _sc_degree(dst)
    deg2 = degs.reshape(NC * NP, 1)
    y1, dinv_b = _tc1(deg2, x_pad, W1)
    aggs1 = _sc_aggregate(y1, src, dst)
    y2 = _tc2(aggs1, y1, dinv_b, row(b1), row(g1), row(be1), row(rm1),
              row(rv1), W2)
    aggs2 = _sc_aggregate(y2, src, dst)
    logp_pad, var2 = _tc3(aggs2, y2, dinv_b, row(b2), row(g2), row(be2),
                          row(rm2), row(rv2), W3p, b3p)
    return (logp_pad[:N, :C], var2[0, 0])


# speedup vs baseline: 28.8923x; 2.2363x over previous
import functools

import jax
import jax.numpy as jnp
from jax import lax
from jax.experimental import pallas as pl
from jax.experimental.pallas import tpu as pltpu
from jax.experimental.pallas import tpu_sc as plsc

N = 10000
E = 320000
D = 128
H = 128
C = 40
BN_EPS = 1e-5

NC = 2
NS = 16
NW = NC * NS
NP = 10240
RPT = NP // NS
EPW = E // NW
CHUNK = 80
NCHUNK = EPW // CHUNK
NBUF = 2
ERODS = E // CHUNK
BLK = 640
GRID = NP // BLK

_mesh = plsc.VectorSubcoreMesh(core_axis_name="c", subcore_axis_name="s")


@functools.partial(
    pl.kernel,
    out_type=jax.ShapeDtypeStruct((NC * NP,), jnp.float32),
    mesh=_mesh,
    scratch_types=[
        pltpu.VMEM_SHARED((NP,), jnp.float32),
        pltpu.VMEM((EPW,), jnp.int32),
        pltpu.VMEM((NBUF, CHUNK), jnp.int32),
        pltpu.VMEM((CHUNK,), jnp.float32),
        pltpu.VMEM((RPT,), jnp.float32),
        pltpu.SemaphoreType.DMA((NBUF,)),
    ],
)
def _sc_degree(dst_hbm, out_hbm, deg_sh, dstiA, dsti_v, ones_v, zeros_v,
               ssem):
    c = lax.axis_index("c")
    s = lax.axis_index("s")
    wid = c * NS + s

    def fill_zero(i, _):
        zeros_v[pl.ds(i * 16, 16)] = jnp.zeros((16,), jnp.float32)
        return _
    lax.fori_loop(0, RPT // 16, fill_zero, None)

    def fill_one(i, _):
        ones_v[pl.ds(i * 16, 16)] = jnp.full((16,), 1.0, jnp.float32)
        return _
    lax.fori_loop(0, CHUNK // 16, fill_one, None)

    pltpu.sync_copy(zeros_v, deg_sh.at[pl.ds(s * RPT, RPT)])
    pltpu.sync_copy(dst_hbm.at[pl.ds(wid * EPW, EPW)], dstiA)
    plsc.subcore_barrier()

    def stage_idx(k, b):
        for j in range(CHUNK // 16):
            dsti_v[b, pl.ds(j * 16, 16)] = dstiA[pl.ds(k * CHUNK + j * 16,
                                                       16)]

    for b in range(NBUF):
        stage_idx(b, b)

    def body(m, _):
        for b in range(NBUF):
            k = m * NBUF + b
            pltpu.async_copy(ones_v, deg_sh.at[dsti_v.at[b]], ssem.at[b],
                             add=True)
            kn = k + NBUF

            @pl.when(kn < NCHUNK)
            def _():
                pltpu.make_async_copy(ones_v, deg_sh.at[dsti_v.at[b]],
                                      ssem.at[b]).wait()
                stage_idx(kn, b)
        return _
    lax.fori_loop(0, NCHUNK // NBUF, body, None)
    for k in range((NCHUNK // NBUF) * NBUF, NCHUNK):
        b = k % NBUF
        pltpu.async_copy(ones_v, deg_sh.at[dsti_v.at[b]], ssem.at[b],
                         add=True)
    for b in range(NBUF):
        pltpu.make_async_copy(ones_v, deg_sh.at[dsti_v.at[b]],
                              ssem.at[b]).wait()

    plsc.subcore_barrier()
    pltpu.sync_copy(deg_sh.at[pl.ds(s * RPT, RPT)],
                    out_hbm.at[pl.ds(c * NP + s * RPT, RPT)])


@functools.partial(
    pl.kernel,
    out_type=jax.ShapeDtypeStruct((NC * NP, H), jnp.float32),
    mesh=_mesh,
    scratch_types=[
        pltpu.VMEM_SHARED((NP, H), jnp.float32),
        pltpu.VMEM((EPW,), jnp.int32),
        pltpu.VMEM((EPW,), jnp.int32),
        pltpu.VMEM((NBUF, CHUNK), jnp.int32),
        pltpu.VMEM((NBUF * CHUNK, H), jnp.float32),
        pltpu.SemaphoreType.DMA((NBUF,)),
        pltpu.SemaphoreType.DMA((NBUF,)),
    ],
)
def _sc_aggregate(y_hbm, src_hbm, dst_hbm, out_hbm,
                  acc_sh, srciA, dstiA, dsti_v, rows_v, gsem, ssem):
    c = lax.axis_index("c")
    s = lax.axis_index("s")
    wid = c * NS + s

    def fill_zero(i, _):
        for j in range(H // 16):
            rows_v[i, pl.ds(j * 16, 16)] = jnp.zeros((16,), jnp.float32)
        return _
    lax.fori_loop(0, 64, fill_zero, None)

    def zero_acc(j, _):
        pltpu.sync_copy(rows_v.at[pl.ds(0, 64)],
                        acc_sh.at[pl.ds(s * RPT + j * 64, 64)])
        return _
    lax.fori_loop(0, RPT // 64, zero_acc, None)

    pltpu.sync_copy(src_hbm.at[pl.ds(wid * EPW, EPW)], srciA)
    pltpu.sync_copy(dst_hbm.at[pl.ds(wid * EPW, EPW)], dstiA)
    plsc.subcore_barrier()

    def stage_dsti(k, b):
        for j in range(CHUNK // 16):
            dsti_v[b, pl.ds(j * 16, 16)] = dstiA[pl.ds(k * CHUNK + j * 16,
                                                       16)]

    def start_gather(k, b):
        pltpu.async_copy(y_hbm.at[srciA.at[pl.ds(k * CHUNK, CHUNK)]],
                         rows_v.at[pl.ds(b * CHUNK, CHUNK)], gsem.at[b])

    def wait_gather(k, b):
        pltpu.make_async_copy(y_hbm.at[srciA.at[pl.ds(k * CHUNK, CHUNK)]],
                              rows_v.at[pl.ds(b * CHUNK, CHUNK)], gsem.at[b]).wait()

    for b in range(NBUF):
        stage_dsti(b, b)
        start_gather(b, b)

    def body(m, _):
        for b in range(NBUF):
            k = m * NBUF + b
            wait_gather(k, b)
            pltpu.async_copy(rows_v.at[pl.ds(b * CHUNK, CHUNK)], acc_sh.at[dsti_v.at[b]],
                             ssem.at[b], add=True)
            kn = k + NBUF

            @pl.when(kn < NCHUNK)
            def _():
                pltpu.make_async_copy(rows_v.at[pl.ds(b * CHUNK, CHUNK)],
                                      acc_sh.at[dsti_v.at[b]],
                                      ssem.at[b]).wait()
                stage_dsti(kn, b)
                start_gather(kn, b)
        return _
    lax.fori_loop(0, NCHUNK // NBUF, body, None)
    for k in range((NCHUNK // NBUF) * NBUF, NCHUNK):
        b = k % NBUF
        wait_gather(k, b)
        pltpu.async_copy(rows_v.at[pl.ds(b * CHUNK, CHUNK)],
                         acc_sh.at[dsti_v.at[b]], ssem.at[b], add=True)
    for b in range(NBUF):
        pltpu.make_async_copy(rows_v.at[pl.ds(b * CHUNK, CHUNK)], acc_sh.at[dsti_v.at[b]],
                              ssem.at[b]).wait()

    plsc.subcore_barrier()
    pltpu.sync_copy(acc_sh.at[pl.ds(s * RPT, RPT)],
                    out_hbm.at[pl.ds(c * NP + s * RPT, RPT)])


def _tc1_body(degA, degB, x_ref, w_ref, y_ref, dinv_ref):
    deg = degA[...] + degB[...] + 1.0
    dinv = lax.rsqrt(deg)
    dinv_ref[...] = jnp.broadcast_to(dinv, (BLK, H))
    h = jnp.dot(x_ref[...], w_ref[...], preferred_element_type=jnp.float32)
    y_ref[...] = h * dinv


def _tc1(deg2, x_pad, W1):
    return pl.pallas_call(
        _tc1_body,
        grid=(GRID,),
        in_specs=[
            pl.BlockSpec((BLK, 1), lambda i: (i, 0)),
            pl.BlockSpec((BLK, 1), lambda i: (i + GRID, 0)),
            pl.BlockSpec((BLK, D), lambda i: (i, 0)),
            pl.BlockSpec((D, H), lambda i: (0, 0)),
        ],
        out_specs=[
            pl.BlockSpec((BLK, H), lambda i: (i, 0)),
            pl.BlockSpec((BLK, H), lambda i: (i, 0)),
        ],
        out_shape=[
            jax.ShapeDtypeStruct((NP, H), jnp.float32),
            jax.ShapeDtypeStruct((NP, H), jnp.float32),
        ],
        compiler_params=pltpu.CompilerParams(
            dimension_semantics=("arbitrary",)),
    )(deg2, deg2, x_pad, W1)


def _tc2_body(aggA, aggB, y_ref, dinv_ref, b_ref, g_ref, be_ref, rm_ref,
              rv_ref, w_ref, out_ref):
    dinv = dinv_ref[...]
    t = dinv * (aggA[...] + aggB[...] + y_ref[...]) + b_ref[...]
    t = jnp.maximum(t, 0.0)
    t = (t - rm_ref[...]) * lax.rsqrt(rv_ref[...] + BN_EPS) * g_ref[...] \
        + be_ref[...]
    t = jnp.maximum(t, 0.0)
    h = jnp.dot(t, w_ref[...], preferred_element_type=jnp.float32)
    out_ref[...] = h * dinv


def _tc2(aggs, y1, dinv_b, b1, g1, be1, rm1, rv1, W2):
    row = pl.BlockSpec((1, H), lambda i: (0, 0))
    return pl.pallas_call(
        _tc2_body,
        grid=(GRID,),
        in_specs=[
            pl.BlockSpec((BLK, H), lambda i: (i, 0)),
            pl.BlockSpec((BLK, H), lambda i: (i + GRID, 0)),
            pl.BlockSpec((BLK, H), lambda i: (i, 0)),
            pl.BlockSpec((BLK, H), lambda i: (i, 0)),
            row, row, row, row, row,
            pl.BlockSpec((H, H), lambda i: (0, 0)),
        ],
        out_specs=pl.BlockSpec((BLK, H), lambda i: (i, 0)),
        out_shape=jax.ShapeDtypeStruct((NP, H), jnp.float32),
        compiler_params=pltpu.CompilerParams(
            dimension_semantics=("arbitrary",)),
    )(aggs, aggs, y1, dinv_b, b1, g1, be1, rm1, rv1, W2)


def _tc3_body(aggA, aggB, y_ref, dinv_ref, b_ref, g_ref, be_ref, rm_ref,
              rv_ref, w_ref, b3_ref, logp_ref, var_ref, acc):
    i = pl.program_id(0)

    @pl.when(i == 0)
    def _():
        acc[0] = 0.0
        acc[1] = 0.0

    t = dinv_ref[...] * (aggA[...] + aggB[...] + y_ref[...]) + b_ref[...]
    t = jnp.maximum(t, 0.0)
    t = (t - rm_ref[...]) * lax.rsqrt(rv_ref[...] + BN_EPS) * g_ref[...] \
        + be_ref[...]
    t = jnp.maximum(t, 0.0)
    o = jnp.dot(t, w_ref[...], preferred_element_type=jnp.float32) + b3_ref[...]

    col = lax.broadcasted_iota(jnp.int32, (BLK, H), 1)
    colmask = col < C
    om = jnp.where(colmask, o, -1e30)
    m = jnp.max(om, axis=1, keepdims=True)
    ex = jnp.where(colmask, jnp.exp(o - m), 0.0)
    se = jnp.sum(ex, axis=1, keepdims=True)
    logp_ref[...] = o - m - jnp.log(se)

    rowg = lax.broadcasted_iota(jnp.int32, (BLK, H), 0) + i * BLK
    vmask = colmask & (rowg < N)
    ov = jnp.where(vmask, o, 0.0)
    acc[0] += jnp.sum(ov)
    acc[1] += jnp.sum(ov * ov)

    n = float(N * C)
    v = (acc[1] - acc[0] * acc[0] / n) / (n - 1.0)
    var_ref[...] = jnp.full((1, 1), v, jnp.float32)


def _tc3(aggs, y2, dinv_b, b2, g2, be2, rm2, rv2, W3p, b3p):
    row = pl.BlockSpec((1, H), lambda i: (0, 0))
    return pl.pallas_call(
        _tc3_body,
        grid=(GRID,),
        in_specs=[
            pl.BlockSpec((BLK, H), lambda i: (i, 0)),
            pl.BlockSpec((BLK, H), lambda i: (i + GRID, 0)),
            pl.BlockSpec((BLK, H), lambda i: (i, 0)),
            pl.BlockSpec((BLK, H), lambda i: (i, 0)),
            row, row, row, row, row,
            pl.BlockSpec((H, H), lambda i: (0, 0)),
            row,
        ],
        out_specs=[
            pl.BlockSpec((BLK, H), lambda i: (i, 0)),
            pl.BlockSpec((1, 1), lambda i: (0, 0)),
        ],
        out_shape=[
            jax.ShapeDtypeStruct((NP, H), jnp.float32),
            jax.ShapeDtypeStruct((1, 1), jnp.float32),
        ],
        scratch_shapes=[pltpu.SMEM((2,), jnp.float32)],
        compiler_params=pltpu.CompilerParams(
            dimension_semantics=("arbitrary",)),
    )(aggs, aggs, y2, dinv_b, b2, g2, be2, rm2, rv2, W3p, b3p)


def kernel(x, edge_index, W1, b1, g1, be1, rm1, rv1, W2, b2, g2, be2, rm2,
           rv2, W3, b3):
    src = edge_index[0]
    dst = edge_index[1]
    x_pad = jnp.pad(x, ((0, NP - N), (0, 0)))

    row = lambda v: v.reshape(1, H)
    W3p = jnp.pad(W3, ((0, 0), (0, H - C)))
    b3p = jnp.pad(b3, (0, H - C)).reshape(1, H)

    degs = _sc_degree(dst)
    deg2 = degs.reshape(NC * NP, 1)
    y1, dinv_b = _tc1(deg2, x_pad, W1)
    aggs1 = _sc_aggregate(y1, src, dst)
    y2 = _tc2(aggs1, y1, dinv_b, row(b1), row(g1), row(be1), row(rm1),
              row(rv1), W2)
    aggs2 = _sc_aggregate(y2, src, dst)
    logp_pad, var2 = _tc3(aggs2, y2, dinv_b, row(b2), row(g2), row(be2),
                          row(rm2), row(rv2), W3p, b3p)
    return (logp_pad[:N, :C], var2[0, 0])


# speedup vs baseline: 31.2433x; 1.0814x over previous
import functools

import jax
import jax.numpy as jnp
from jax import lax
from jax.experimental import pallas as pl
from jax.experimental.pallas import tpu as pltpu
from jax.experimental.pallas import tpu_sc as plsc

N = 10000
E = 320000
D = 128
H = 128
C = 40
BN_EPS = 1e-5

NC = 2
NS = 16
NW = NC * NS
NP = 10240
RPT = NP // NS
EPW = E // NW
CHUNK = 80
NCHUNK = EPW // CHUNK
NBUF = 2
ERODS = E // CHUNK
BLK = 640
GRID = NP // BLK

_mesh = plsc.VectorSubcoreMesh(core_axis_name="c", subcore_axis_name="s")


@functools.partial(
    pl.kernel,
    out_type=jax.ShapeDtypeStruct((NC * NP,), jnp.float32),
    mesh=_mesh,
    scratch_types=[
        pltpu.VMEM_SHARED((NP,), jnp.float32),
        pltpu.VMEM((EPW,), jnp.int32),
        pltpu.VMEM((NBUF, CHUNK), jnp.int32),
        pltpu.VMEM((CHUNK,), jnp.float32),
        pltpu.VMEM((RPT,), jnp.float32),
        pltpu.SemaphoreType.DMA((NBUF,)),
    ],
)
def _sc_degree(dst_hbm, out_hbm, deg_sh, dstiA, dsti_v, ones_v, zeros_v,
               ssem):
    c = lax.axis_index("c")
    s = lax.axis_index("s")
    wid = c * NS + s

    def fill_zero(i, _):
        zeros_v[pl.ds(i * 16, 16)] = jnp.zeros((16,), jnp.float32)
        return _
    lax.fori_loop(0, RPT // 16, fill_zero, None)

    def fill_one(i, _):
        ones_v[pl.ds(i * 16, 16)] = jnp.full((16,), 1.0, jnp.float32)
        return _
    lax.fori_loop(0, CHUNK // 16, fill_one, None)

    pltpu.sync_copy(zeros_v, deg_sh.at[pl.ds(s * RPT, RPT)])
    pltpu.sync_copy(dst_hbm.at[pl.ds(wid * EPW, EPW)], dstiA)
    plsc.subcore_barrier()

    def stage_idx(k, b):
        for j in range(CHUNK // 16):
            dsti_v[b, pl.ds(j * 16, 16)] = dstiA[pl.ds(k * CHUNK + j * 16,
                                                       16)]

    for b in range(NBUF):
        stage_idx(b, b)

    def body(m, _):
        for b in range(NBUF):
            k = m * NBUF + b
            pltpu.async_copy(ones_v, deg_sh.at[dsti_v.at[b]], ssem.at[b],
                             add=True)
            kn = k + NBUF

            @pl.when(kn < NCHUNK)
            def _():
                pltpu.make_async_copy(ones_v, deg_sh.at[dsti_v.at[b]],
                                      ssem.at[b]).wait()
                stage_idx(kn, b)
        return _
    lax.fori_loop(0, NCHUNK // NBUF, body, None)
    for k in range((NCHUNK // NBUF) * NBUF, NCHUNK):
        b = k % NBUF
        pltpu.async_copy(ones_v, deg_sh.at[dsti_v.at[b]], ssem.at[b],
                         add=True)
    for b in range(NBUF):
        pltpu.make_async_copy(ones_v, deg_sh.at[dsti_v.at[b]],
                              ssem.at[b]).wait()

    plsc.subcore_barrier()
    pltpu.sync_copy(deg_sh.at[pl.ds(s * RPT, RPT)],
                    out_hbm.at[pl.ds(c * NP + s * RPT, RPT)])


ACH = 128
AFULL = EPW // ACH
AREM = EPW - AFULL * ACH
ANB = 2


@functools.partial(
    pl.kernel,
    out_type=jax.ShapeDtypeStruct((NC * NP, H), jnp.float32),
    mesh=_mesh,
    scratch_types=[
        pltpu.VMEM_SHARED((NP, H), jnp.float32),
        pltpu.VMEM((EPW,), jnp.int32),
        pltpu.VMEM((ANB, ACH), jnp.int32),
        pltpu.VMEM((AREM,), jnp.int32),
        pltpu.VMEM((ANB * ACH, H), jnp.float32),
        pltpu.SemaphoreType.DMA((ANB,)),
        pltpu.SemaphoreType.DMA((ANB,)),
        pltpu.SemaphoreType.DMA((ANB,)),
    ],
)
def _sc_aggregate(y_hbm, src_hbm, dst_hbm, out_hbm,
                  acc_sh, srciA, dsti_v, dsti_r, rows_v, gsem, ssem, isem):
    c = lax.axis_index("c")
    s = lax.axis_index("s")
    wid = c * NS + s

    def fill_zero(i, _):
        for j in range(H // 16):
            rows_v[i, pl.ds(j * 16, 16)] = jnp.zeros((16,), jnp.float32)
        return _
    lax.fori_loop(0, 64, fill_zero, None)

    def zero_acc(j, _):
        pltpu.sync_copy(rows_v.at[pl.ds(0, 64)],
                        acc_sh.at[pl.ds(s * RPT + j * 64, 64)])
        return _
    lax.fori_loop(0, RPT // 64, zero_acc, None)

    pltpu.sync_copy(src_hbm.at[pl.ds(wid * EPW, EPW)], srciA)
    plsc.subcore_barrier()

    def d_copy(k, b):
        return pltpu.make_async_copy(
            dst_hbm.at[pl.ds(wid * EPW + k * ACH, ACH)],
            dsti_v.at[b], isem.at[b])

    def g_copy(k, b):
        return pltpu.make_async_copy(
            y_hbm.at[srciA.at[pl.ds(k * ACH, ACH)]],
            rows_v.at[pl.ds(b * ACH, ACH)], gsem.at[b])

    def s_copy(b):
        return pltpu.make_async_copy(
            rows_v.at[pl.ds(b * ACH, ACH)],
            acc_sh.at[dsti_v.at[b]], ssem.at[b])

    for b in range(ANB):
        d_copy(b, b).start()
        g_copy(b, b).start()

    def body(m, _):
        for b in range(ANB):
            k = m * ANB + b
            g_copy(k, b).wait()
            d_copy(k, b).wait()
            pltpu.async_copy(rows_v.at[pl.ds(b * ACH, ACH)],
                             acc_sh.at[dsti_v.at[b]], ssem.at[b], add=True)
            s_copy(b).wait()
            kn = k + ANB

            @pl.when(kn < AFULL)
            def _():
                d_copy(kn, b).start()
                g_copy(kn, b).start()
        return _
    lax.fori_loop(0, AFULL // ANB, body, None)

    rem = AFULL * ACH
    pltpu.sync_copy(dst_hbm.at[pl.ds(wid * EPW + rem, AREM)], dsti_r)
    pltpu.async_copy(y_hbm.at[srciA.at[pl.ds(rem, AREM)]],
                     rows_v.at[pl.ds(0, AREM)], gsem.at[0])
    pltpu.make_async_copy(y_hbm.at[srciA.at[pl.ds(rem, AREM)]],
                          rows_v.at[pl.ds(0, AREM)], gsem.at[0]).wait()
    pltpu.sync_copy(rows_v.at[pl.ds(0, AREM)], acc_sh.at[dsti_r], add=True)

    plsc.subcore_barrier()
    pltpu.sync_copy(acc_sh.at[pl.ds(s * RPT, RPT)],
                    out_hbm.at[pl.ds(c * NP + s * RPT, RPT)])


def _tc1_body(degA, degB, x_ref, w_ref, y_ref, dinv_ref):
    deg = degA[...] + degB[...] + 1.0
    dinv = lax.rsqrt(deg)
    dinv_ref[...] = jnp.broadcast_to(dinv, (BLK, H))
    h = jnp.dot(x_ref[...], w_ref[...], preferred_element_type=jnp.float32)
    y_ref[...] = h * dinv


def _tc1(deg2, x_pad, W1):
    return pl.pallas_call(
        _tc1_body,
        grid=(GRID,),
        in_specs=[
            pl.BlockSpec((BLK, 1), lambda i: (i, 0)),
            pl.BlockSpec((BLK, 1), lambda i: (i + GRID, 0)),
            pl.BlockSpec((BLK, D), lambda i: (i, 0)),
            pl.BlockSpec((D, H), lambda i: (0, 0)),
        ],
        out_specs=[
            pl.BlockSpec((BLK, H), lambda i: (i, 0)),
            pl.BlockSpec((BLK, H), lambda i: (i, 0)),
        ],
        out_shape=[
            jax.ShapeDtypeStruct((NP, H), jnp.float32),
            jax.ShapeDtypeStruct((NP, H), jnp.float32),
        ],
        compiler_params=pltpu.CompilerParams(
            dimension_semantics=("arbitrary",)),
    )(deg2, deg2, x_pad, W1)


def _tc2_body(aggA, aggB, y_ref, dinv_ref, b_ref, g_ref, be_ref, rm_ref,
              rv_ref, w_ref, out_ref):
    dinv = dinv_ref[...]
    t = dinv * (aggA[...] + aggB[...] + y_ref[...]) + b_ref[...]
    t = jnp.maximum(t, 0.0)
    t = (t - rm_ref[...]) * lax.rsqrt(rv_ref[...] + BN_EPS) * g_ref[...] \
        + be_ref[...]
    t = jnp.maximum(t, 0.0)
    h = jnp.dot(t, w_ref[...], preferred_element_type=jnp.float32)
    out_ref[...] = h * dinv


def _tc2(aggs, y1, dinv_b, b1, g1, be1, rm1, rv1, W2):
    row = pl.BlockSpec((1, H), lambda i: (0, 0))
    return pl.pallas_call(
        _tc2_body,
        grid=(GRID,),
        in_specs=[
            pl.BlockSpec((BLK, H), lambda i: (i, 0)),
            pl.BlockSpec((BLK, H), lambda i: (i + GRID, 0)),
            pl.BlockSpec((BLK, H), lambda i: (i, 0)),
            pl.BlockSpec((BLK, H), lambda i: (i, 0)),
            row, row, row, row, row,
            pl.BlockSpec((H, H), lambda i: (0, 0)),
        ],
        out_specs=pl.BlockSpec((BLK, H), lambda i: (i, 0)),
        out_shape=jax.ShapeDtypeStruct((NP, H), jnp.float32),
        compiler_params=pltpu.CompilerParams(
            dimension_semantics=("arbitrary",)),
    )(aggs, aggs, y1, dinv_b, b1, g1, be1, rm1, rv1, W2)


def _tc3_body(aggA, aggB, y_ref, dinv_ref, b_ref, g_ref, be_ref, rm_ref,
              rv_ref, w_ref, b3_ref, logp_ref, var_ref, acc):
    i = pl.program_id(0)

    @pl.when(i == 0)
    def _():
        acc[0] = 0.0
        acc[1] = 0.0

    t = dinv_ref[...] * (aggA[...] + aggB[...] + y_ref[...]) + b_ref[...]
    t = jnp.maximum(t, 0.0)
    t = (t - rm_ref[...]) * lax.rsqrt(rv_ref[...] + BN_EPS) * g_ref[...] \
        + be_ref[...]
    t = jnp.maximum(t, 0.0)
    o = jnp.dot(t, w_ref[...], preferred_element_type=jnp.float32) + b3_ref[...]

    col = lax.broadcasted_iota(jnp.int32, (BLK, H), 1)
    colmask = col < C
    om = jnp.where(colmask, o, -1e30)
    m = jnp.max(om, axis=1, keepdims=True)
    ex = jnp.where(colmask, jnp.exp(o - m), 0.0)
    se = jnp.sum(ex, axis=1, keepdims=True)
    logp_ref[...] = o - m - jnp.log(se)

    rowg = lax.broadcasted_iota(jnp.int32, (BLK, H), 0) + i * BLK
    vmask = colmask & (rowg < N)
    ov = jnp.where(vmask, o, 0.0)
    acc[0] += jnp.sum(ov)
    acc[1] += jnp.sum(ov * ov)

    n = float(N * C)
    v = (acc[1] - acc[0] * acc[0] / n) / (n - 1.0)
    var_ref[...] = jnp.full((1, 1), v, jnp.float32)


def _tc3(aggs, y2, dinv_b, b2, g2, be2, rm2, rv2, W3p, b3p):
    row = pl.BlockSpec((1, H), lambda i: (0, 0))
    return pl.pallas_call(
        _tc3_body,
        grid=(GRID,),
        in_specs=[
            pl.BlockSpec((BLK, H), lambda i: (i, 0)),
            pl.BlockSpec((BLK, H), lambda i: (i + GRID, 0)),
            pl.BlockSpec((BLK, H), lambda i: (i, 0)),
            pl.BlockSpec((BLK, H), lambda i: (i, 0)),
            row, row, row, row, row,
            pl.BlockSpec((H, H), lambda i: (0, 0)),
            row,
        ],
        out_specs=[
            pl.BlockSpec((BLK, H), lambda i: (i, 0)),
            pl.BlockSpec((1, 1), lambda i: (0, 0)),
        ],
        out_shape=[
            jax.ShapeDtypeStruct((NP, H), jnp.float32),
            jax.ShapeDtypeStruct((1, 1), jnp.float32),
        ],
        scratch_shapes=[pltpu.SMEM((2,), jnp.float32)],
        compiler_params=pltpu.CompilerParams(
            dimension_semantics=("arbitrary",)),
    )(aggs, aggs, y2, dinv_b, b2, g2, be2, rm2, rv2, W3p, b3p)


def kernel(x, edge_index, W1, b1, g1, be1, rm1, rv1, W2, b2, g2, be2, rm2,
           rv2, W3, b3):
    src = edge_index[0]
    dst = edge_index[1]
    x_pad = jnp.pad(x, ((0, NP - N), (0, 0)))

    row = lambda v: v.reshape(1, H)
    W3p = jnp.pad(W3, ((0, 0), (0, H - C)))
    b3p = jnp.pad(b3, (0, H - C)).reshape(1, H)

    degs = _sc_degree(dst)
    deg2 = degs.reshape(NC * NP, 1)
    y1, dinv_b = _tc1(deg2, x_pad, W1)
    aggs1 = _sc_aggregate(y1, src, dst)
    y2 = _tc2(aggs1, y1, dinv_b, row(b1), row(g1), row(be1), row(rm1),
              row(rv1), W2)
    aggs2 = _sc_aggregate(y2, src, dst)
    logp_pad, var2 = _tc3(aggs2, y2, dinv_b, row(b2), row(g2), row(be2),
                          row(rm2), row(rv2), W3p, b3p)
    return (logp_pad[:N, :C], var2[0, 0])


# speedup vs baseline: 31.2665x; 1.0007x over previous
import functools

import jax
import jax.numpy as jnp
from jax import lax
from jax.experimental import pallas as pl
from jax.experimental.pallas import tpu as pltpu
from jax.experimental.pallas import tpu_sc as plsc

N = 10000
E = 320000
D = 128
H = 128
C = 40
BN_EPS = 1e-5

NC = 2
NS = 16
NW = NC * NS
NP = 10240
RPT = NP // NS
EPW = E // NW
CHUNK = 80
NCHUNK = EPW // CHUNK
NBUF = 2
ERODS = E // CHUNK
BLK = 640
GRID = NP // BLK

_mesh = plsc.VectorSubcoreMesh(core_axis_name="c", subcore_axis_name="s")


DCH = 128
DFULL = EPW // DCH
DREM = EPW - DFULL * DCH
DNB = 2


@functools.partial(
    pl.kernel,
    out_type=jax.ShapeDtypeStruct((NC * NP,), jnp.float32),
    mesh=_mesh,
    scratch_types=[
        pltpu.VMEM_SHARED((NP,), jnp.float32),
        pltpu.VMEM((EPW,), jnp.int32),
        pltpu.VMEM((DNB, DCH), jnp.int32),
        pltpu.VMEM((DREM,), jnp.int32),
        pltpu.VMEM((DCH,), jnp.float32),
        pltpu.VMEM((RPT,), jnp.float32),
        pltpu.SemaphoreType.DMA((DNB,)),
    ],
)
def _sc_degree(dst_hbm, out_hbm, deg_sh, dstiA, dsti_v, dsti_r, ones_v,
               zeros_v, ssem):
    c = lax.axis_index("c")
    s = lax.axis_index("s")
    wid = c * NS + s

    def fill_zero(i, _):
        zeros_v[pl.ds(i * 16, 16)] = jnp.zeros((16,), jnp.float32)
        return _
    lax.fori_loop(0, RPT // 16, fill_zero, None)

    def fill_one(i, _):
        ones_v[pl.ds(i * 16, 16)] = jnp.full((16,), 1.0, jnp.float32)
        return _
    lax.fori_loop(0, DCH // 16, fill_one, None)

    pltpu.sync_copy(zeros_v, deg_sh.at[pl.ds(s * RPT, RPT)])
    pltpu.sync_copy(dst_hbm.at[pl.ds(wid * EPW, EPW)], dstiA)
    plsc.subcore_barrier()

    def stage_idx(k, b):
        for j in range(DCH // 16):
            dsti_v[b, pl.ds(j * 16, 16)] = dstiA[pl.ds(k * DCH + j * 16, 16)]

    for b in range(DNB):
        stage_idx(b, b)

    def body(m, _):
        for b in range(DNB):
            k = m * DNB + b
            pltpu.async_copy(ones_v, deg_sh.at[dsti_v.at[b]], ssem.at[b],
                             add=True)
            kn = k + DNB

            @pl.when(kn < DFULL)
            def _():
                pltpu.make_async_copy(ones_v, deg_sh.at[dsti_v.at[b]],
                                      ssem.at[b]).wait()
                stage_idx(kn, b)
        return _
    lax.fori_loop(0, DFULL // DNB, body, None)
    for b in range(DNB):
        pltpu.make_async_copy(ones_v, deg_sh.at[dsti_v.at[b]],
                              ssem.at[b]).wait()
    for j in range(DREM // 16):
        dsti_r[pl.ds(j * 16, 16)] = dstiA[pl.ds(DFULL * DCH + j * 16, 16)]
    pltpu.sync_copy(ones_v.at[pl.ds(0, DREM)], deg_sh.at[dsti_r], add=True)

    plsc.subcore_barrier()
    pltpu.sync_copy(deg_sh.at[pl.ds(s * RPT, RPT)],
                    out_hbm.at[pl.ds(c * NP + s * RPT, RPT)])


ACH = 128
AFULL = EPW // ACH
AREM = EPW - AFULL * ACH
ANB = 2


@functools.partial(
    pl.kernel,
    out_type=jax.ShapeDtypeStruct((NC * NP, H), jnp.float32),
    mesh=_mesh,
    scratch_types=[
        pltpu.VMEM_SHARED((NP, H), jnp.float32),
        pltpu.VMEM((EPW,), jnp.int32),
        pltpu.VMEM((ANB, ACH), jnp.int32),
        pltpu.VMEM((AREM,), jnp.int32),
        pltpu.VMEM((ANB * ACH, H), jnp.float32),
        pltpu.SemaphoreType.DMA((ANB,)),
        pltpu.SemaphoreType.DMA((ANB,)),
        pltpu.SemaphoreType.DMA((ANB,)),
    ],
)
def _sc_aggregate(y_hbm, src_hbm, dst_hbm, out_hbm,
                  acc_sh, srciA, dsti_v, dsti_r, rows_v, gsem, ssem, isem):
    c = lax.axis_index("c")
    s = lax.axis_index("s")
    wid = c * NS + s

    def fill_zero(i, _):
        for j in range(H // 16):
            rows_v[i, pl.ds(j * 16, 16)] = jnp.zeros((16,), jnp.float32)
        return _
    lax.fori_loop(0, 64, fill_zero, None)

    def zero_acc(j, _):
        pltpu.sync_copy(rows_v.at[pl.ds(0, 64)],
                        acc_sh.at[pl.ds(s * RPT + j * 64, 64)])
        return _
    lax.fori_loop(0, RPT // 64, zero_acc, None)

    pltpu.sync_copy(src_hbm.at[pl.ds(wid * EPW, EPW)], srciA)
    plsc.subcore_barrier()

    def d_copy(k, b):
        return pltpu.make_async_copy(
            dst_hbm.at[pl.ds(wid * EPW + k * ACH, ACH)],
            dsti_v.at[b], isem.at[b])

    def g_copy(k, b):
        return pltpu.make_async_copy(
            y_hbm.at[srciA.at[pl.ds(k * ACH, ACH)]],
            rows_v.at[pl.ds(b * ACH, ACH)], gsem.at[b])

    def s_copy(b):
        return pltpu.make_async_copy(
            rows_v.at[pl.ds(b * ACH, ACH)],
            acc_sh.at[dsti_v.at[b]], ssem.at[b])

    for b in range(ANB):
        d_copy(b, b).start()
        g_copy(b, b).start()

    def body(m, _):
        for b in range(ANB):
            k = m * ANB + b
            g_copy(k, b).wait()
            d_copy(k, b).wait()
            pltpu.async_copy(rows_v.at[pl.ds(b * ACH, ACH)],
                             acc_sh.at[dsti_v.at[b]], ssem.at[b], add=True)
            s_copy(b).wait()
            kn = k + ANB

            @pl.when(kn < AFULL)
            def _():
                d_copy(kn, b).start()
                g_copy(kn, b).start()
        return _
    lax.fori_loop(0, AFULL // ANB, body, None)

    rem = AFULL * ACH
    pltpu.sync_copy(dst_hbm.at[pl.ds(wid * EPW + rem, AREM)], dsti_r)
    pltpu.async_copy(y_hbm.at[srciA.at[pl.ds(rem, AREM)]],
                     rows_v.at[pl.ds(0, AREM)], gsem.at[0])
    pltpu.make_async_copy(y_hbm.at[srciA.at[pl.ds(rem, AREM)]],
                          rows_v.at[pl.ds(0, AREM)], gsem.at[0]).wait()
    pltpu.sync_copy(rows_v.at[pl.ds(0, AREM)], acc_sh.at[dsti_r], add=True)

    plsc.subcore_barrier()
    pltpu.sync_copy(acc_sh.at[pl.ds(s * RPT, RPT)],
                    out_hbm.at[pl.ds(c * NP + s * RPT, RPT)])


def _tc1_body(degA, degB, x_ref, w_ref, y_ref):
    deg = degA[...] + degB[...] + 1.0
    dinv = lax.rsqrt(deg)
    h = jnp.dot(x_ref[...], w_ref[...], preferred_element_type=jnp.float32)
    y_ref[...] = h * dinv


def _tc1(deg2, x_pad, W1):
    return pl.pallas_call(
        _tc1_body,
        grid=(GRID,),
        in_specs=[
            pl.BlockSpec((BLK, 1), lambda i: (i, 0)),
            pl.BlockSpec((BLK, 1), lambda i: (i + GRID, 0)),
            pl.BlockSpec((BLK, D), lambda i: (i, 0)),
            pl.BlockSpec((D, H), lambda i: (0, 0)),
        ],
        out_specs=pl.BlockSpec((BLK, H), lambda i: (i, 0)),
        out_shape=jax.ShapeDtypeStruct((NP, H), jnp.float32),
        compiler_params=pltpu.CompilerParams(
            dimension_semantics=("arbitrary",)),
    )(deg2, deg2, x_pad, W1)


def _tc2_body(degA, degB, aggA, aggB, y_ref, b_ref, g_ref, be_ref, rm_ref,
              rv_ref, w_ref, out_ref):
    dinv = lax.rsqrt(degA[...] + degB[...] + 1.0)
    t = dinv * (aggA[...] + aggB[...] + y_ref[...]) + b_ref[...]
    t = jnp.maximum(t, 0.0)
    t = (t - rm_ref[...]) * lax.rsqrt(rv_ref[...] + BN_EPS) * g_ref[...] \
        + be_ref[...]
    t = jnp.maximum(t, 0.0)
    h = jnp.dot(t, w_ref[...], preferred_element_type=jnp.float32)
    out_ref[...] = h * dinv


def _tc2(deg2, aggs, y1, b1, g1, be1, rm1, rv1, W2):
    row = pl.BlockSpec((1, H), lambda i: (0, 0))
    return pl.pallas_call(
        _tc2_body,
        grid=(GRID,),
        in_specs=[
            pl.BlockSpec((BLK, 1), lambda i: (i, 0)),
            pl.BlockSpec((BLK, 1), lambda i: (i + GRID, 0)),
            pl.BlockSpec((BLK, H), lambda i: (i, 0)),
            pl.BlockSpec((BLK, H), lambda i: (i + GRID, 0)),
            pl.BlockSpec((BLK, H), lambda i: (i, 0)),
            row, row, row, row, row,
            pl.BlockSpec((H, H), lambda i: (0, 0)),
        ],
        out_specs=pl.BlockSpec((BLK, H), lambda i: (i, 0)),
        out_shape=jax.ShapeDtypeStruct((NP, H), jnp.float32),
        compiler_params=pltpu.CompilerParams(
            dimension_semantics=("arbitrary",)),
    )(deg2, deg2, aggs, aggs, y1, b1, g1, be1, rm1, rv1, W2)


def _tc3_body(degA, degB, aggA, aggB, y_ref, b_ref, g_ref, be_ref, rm_ref,
              rv_ref, w_ref, b3_ref, logp_ref, var_ref, acc):
    i = pl.program_id(0)

    @pl.when(i == 0)
    def _():
        acc[0] = 0.0
        acc[1] = 0.0

    dinv = lax.rsqrt(degA[...] + degB[...] + 1.0)
    t = dinv * (aggA[...] + aggB[...] + y_ref[...]) + b_ref[...]
    t = jnp.maximum(t, 0.0)
    t = (t - rm_ref[...]) * lax.rsqrt(rv_ref[...] + BN_EPS) * g_ref[...] \
        + be_ref[...]
    t = jnp.maximum(t, 0.0)
    o = jnp.dot(t, w_ref[...], preferred_element_type=jnp.float32) + b3_ref[...]

    col = lax.broadcasted_iota(jnp.int32, (BLK, H), 1)
    colmask = col < C
    om = jnp.where(colmask, o, -1e30)
    m = jnp.max(om, axis=1, keepdims=True)
    ex = jnp.where(colmask, jnp.exp(o - m), 0.0)
    se = jnp.sum(ex, axis=1, keepdims=True)
    logp_ref[...] = o - m - jnp.log(se)

    rowg = lax.broadcasted_iota(jnp.int32, (BLK, H), 0) + i * BLK
    vmask = colmask & (rowg < N)
    ov = jnp.where(vmask, o, 0.0)
    acc[0] += jnp.sum(ov)
    acc[1] += jnp.sum(ov * ov)

    n = float(N * C)
    v = (acc[1] - acc[0] * acc[0] / n) / (n - 1.0)
    var_ref[...] = jnp.full((1, 1), v, jnp.float32)


def _tc3(deg2, aggs, y2, b2, g2, be2, rm2, rv2, W3p, b3p):
    row = pl.BlockSpec((1, H), lambda i: (0, 0))
    return pl.pallas_call(
        _tc3_body,
        grid=(GRID,),
        in_specs=[
            pl.BlockSpec((BLK, 1), lambda i: (i, 0)),
            pl.BlockSpec((BLK, 1), lambda i: (i + GRID, 0)),
            pl.BlockSpec((BLK, H), lambda i: (i, 0)),
            pl.BlockSpec((BLK, H), lambda i: (i + GRID, 0)),
            pl.BlockSpec((BLK, H), lambda i: (i, 0)),
            row, row, row, row, row,
            pl.BlockSpec((H, H), lambda i: (0, 0)),
            row,
        ],
        out_specs=[
            pl.BlockSpec((BLK, H), lambda i: (i, 0)),
            pl.BlockSpec((1, 1), lambda i: (0, 0)),
        ],
        out_shape=[
            jax.ShapeDtypeStruct((NP, H), jnp.float32),
            jax.ShapeDtypeStruct((1, 1), jnp.float32),
        ],
        scratch_shapes=[pltpu.SMEM((2,), jnp.float32)],
        compiler_params=pltpu.CompilerParams(
            dimension_semantics=("arbitrary",)),
    )(deg2, deg2, aggs, aggs, y2, b2, g2, be2, rm2, rv2, W3p, b3p)


def kernel(x, edge_index, W1, b1, g1, be1, rm1, rv1, W2, b2, g2, be2, rm2,
           rv2, W3, b3):
    src = edge_index[0]
    dst = edge_index[1]
    x_pad = jnp.pad(x, ((0, NP - N), (0, 0)))

    row = lambda v: v.reshape(1, H)
    W3p = jnp.pad(W3, ((0, 0), (0, H - C)))
    b3p = jnp.pad(b3, (0, H - C)).reshape(1, H)

    degs = _sc_degree(dst)
    deg2 = degs.reshape(NC * NP, 1)
    y1 = _tc1(deg2, x_pad, W1)
    aggs1 = _sc_aggregate(y1, src, dst)
    y2 = _tc2(deg2, aggs1, y1, row(b1), row(g1), row(be1), row(rm1),
              row(rv1), W2)
    aggs2 = _sc_aggregate(y2, src, dst)
    logp_pad, var2 = _tc3(deg2, aggs2, y2, row(b2), row(g2), row(be2),
                          row(rm2), row(rv2), W3p, b3p)
    return (logp_pad[:N, :C], var2[0, 0])


# speedup vs baseline: 31.2707x; 1.0001x over previous
import functools

import jax
import jax.numpy as jnp
from jax import lax
from jax.experimental import pallas as pl
from jax.experimental.pallas import tpu as pltpu
from jax.experimental.pallas import tpu_sc as plsc

N = 10000
E = 320000
D = 128
H = 128
C = 40
BN_EPS = 1e-5

NC = 2
NS = 16
NW = NC * NS
NP = 10240
RPT = NP // NS
EPW = E // NW
BLK = 640
GRID = NP // BLK

_mesh = plsc.VectorSubcoreMesh(core_axis_name="c", subcore_axis_name="s")


DCH = 128
DFULL = EPW // DCH
DREM = EPW - DFULL * DCH
DNB = 2


@functools.partial(
    pl.kernel,
    out_type=jax.ShapeDtypeStruct((NC * NP,), jnp.float32),
    mesh=_mesh,
    scratch_types=[
        pltpu.VMEM_SHARED((NP,), jnp.float32),
        pltpu.VMEM((EPW,), jnp.int32),
        pltpu.VMEM((DNB, DCH), jnp.int32),
        pltpu.VMEM((DREM,), jnp.int32),
        pltpu.VMEM((DCH,), jnp.float32),
        pltpu.VMEM((RPT,), jnp.float32),
        pltpu.SemaphoreType.DMA((DNB,)),
    ],
)
def _sc_degree(dst_hbm, out_hbm, deg_sh, dstiA, dsti_v, dsti_r, ones_v,
               zeros_v, ssem):
    c = lax.axis_index("c")
    s = lax.axis_index("s")
    wid = c * NS + s

    def fill_zero(i, _):
        zeros_v[pl.ds(i * 16, 16)] = jnp.zeros((16,), jnp.float32)
        return _
    lax.fori_loop(0, RPT // 16, fill_zero, None)

    def fill_one(i, _):
        ones_v[pl.ds(i * 16, 16)] = jnp.full((16,), 1.0, jnp.float32)
        return _
    lax.fori_loop(0, DCH // 16, fill_one, None)

    pltpu.sync_copy(zeros_v, deg_sh.at[pl.ds(s * RPT, RPT)])
    pltpu.sync_copy(dst_hbm.at[pl.ds(wid * EPW, EPW)], dstiA)
    plsc.subcore_barrier()

    def stage_idx(k, b):
        for j in range(DCH // 16):
            dsti_v[b, pl.ds(j * 16, 16)] = dstiA[pl.ds(k * DCH + j * 16, 16)]

    for b in range(DNB):
        stage_idx(b, b)

    def body(m, _):
        for b in range(DNB):
            k = m * DNB + b
            pltpu.async_copy(ones_v, deg_sh.at[dsti_v.at[b]], ssem.at[b],
                             add=True)
            kn = k + DNB

            @pl.when(kn < DFULL)
            def _():
                pltpu.make_async_copy(ones_v, deg_sh.at[dsti_v.at[b]],
                                      ssem.at[b]).wait()
                stage_idx(kn, b)
        return _
    lax.fori_loop(0, DFULL // DNB, body, None)
    for b in range(DNB):
        pltpu.make_async_copy(ones_v, deg_sh.at[dsti_v.at[b]],
                              ssem.at[b]).wait()
    for j in range(DREM // 16):
        dsti_r[pl.ds(j * 16, 16)] = dstiA[pl.ds(DFULL * DCH + j * 16, 16)]
    pltpu.sync_copy(ones_v.at[pl.ds(0, DREM)], deg_sh.at[dsti_r], add=True)

    plsc.subcore_barrier()
    pltpu.sync_copy(deg_sh.at[pl.ds(s * RPT, RPT)],
                    out_hbm.at[pl.ds(c * NP + s * RPT, RPT)])


ACH = 128
AFULL = EPW // ACH
AREM = EPW - AFULL * ACH
ANB = 2


@functools.partial(
    pl.kernel,
    out_type=jax.ShapeDtypeStruct((NC * NP, H), jnp.float32),
    mesh=_mesh,
    scratch_types=[
        pltpu.VMEM_SHARED((NP, H), jnp.float32),
        pltpu.VMEM((EPW,), jnp.int32),
        pltpu.VMEM((ANB, ACH), jnp.int32),
        pltpu.VMEM((AREM,), jnp.int32),
        pltpu.VMEM((ANB * ACH, H), jnp.float32),
        pltpu.SemaphoreType.DMA((ANB,)),
        pltpu.SemaphoreType.DMA((ANB,)),
        pltpu.SemaphoreType.DMA((ANB,)),
    ],
)
def _sc_aggregate(y_hbm, src_hbm, dst_hbm, out_hbm,
                  acc_sh, srciA, dsti_v, dsti_r, rows_v, gsem, ssem, isem):
    c = lax.axis_index("c")
    s = lax.axis_index("s")
    wid = c * NS + s

    def fill_zero(i, _):
        for j in range(H // 16):
            rows_v[i, pl.ds(j * 16, 16)] = jnp.zeros((16,), jnp.float32)
        return _
    lax.fori_loop(0, 64, fill_zero, None)

    def zero_acc(j, _):
        pltpu.sync_copy(rows_v.at[pl.ds(0, 64)],
                        acc_sh.at[pl.ds(s * RPT + j * 64, 64)])
        return _
    lax.fori_loop(0, RPT // 64, zero_acc, None)

    pltpu.sync_copy(src_hbm.at[pl.ds(wid * EPW, EPW)], srciA)
    plsc.subcore_barrier()

    def d_copy(k, b):
        return pltpu.make_async_copy(
            dst_hbm.at[pl.ds(wid * EPW + k * ACH, ACH)],
            dsti_v.at[b], isem.at[b])

    def g_copy(k, b):
        return pltpu.make_async_copy(
            y_hbm.at[srciA.at[pl.ds(k * ACH, ACH)]],
            rows_v.at[pl.ds(b * ACH, ACH)], gsem.at[b])

    def s_copy(b):
        return pltpu.make_async_copy(
            rows_v.at[pl.ds(b * ACH, ACH)],
            acc_sh.at[dsti_v.at[b]], ssem.at[b])

    for b in range(ANB):
        d_copy(b, b).start()
        g_copy(b, b).start()

    def body(m, _):
        for b in range(ANB):
            k = m * ANB + b
            g_copy(k, b).wait()
            d_copy(k, b).wait()
            pltpu.async_copy(rows_v.at[pl.ds(b * ACH, ACH)],
                             acc_sh.at[dsti_v.at[b]], ssem.at[b], add=True)
            s_copy(b).wait()
            kn = k + ANB

            @pl.when(kn < AFULL)
            def _():
                d_copy(kn, b).start()
                g_copy(kn, b).start()
        return _
    lax.fori_loop(0, AFULL // ANB, body, None)

    rem = AFULL * ACH
    pltpu.sync_copy(dst_hbm.at[pl.ds(wid * EPW + rem, AREM)], dsti_r)
    pltpu.async_copy(y_hbm.at[srciA.at[pl.ds(rem, AREM)]],
                     rows_v.at[pl.ds(0, AREM)], gsem.at[0])
    pltpu.make_async_copy(y_hbm.at[srciA.at[pl.ds(rem, AREM)]],
                          rows_v.at[pl.ds(0, AREM)], gsem.at[0]).wait()
    pltpu.sync_copy(rows_v.at[pl.ds(0, AREM)], acc_sh.at[dsti_r], add=True)

    plsc.subcore_barrier()
    pltpu.sync_copy(acc_sh.at[pl.ds(s * RPT, RPT)],
                    out_hbm.at[pl.ds(c * NP + s * RPT, RPT)])


def _tc1_body(degA, degB, x_ref, w_ref, y_ref):
    deg = degA[...] + degB[...] + 1.0
    dinv = lax.rsqrt(deg)
    h = jnp.dot(x_ref[...], w_ref[...], preferred_element_type=jnp.float32)
    y_ref[...] = h * dinv


def _tc1(deg2, x_pad, W1):
    return pl.pallas_call(
        _tc1_body,
        grid=(GRID,),
        in_specs=[
            pl.BlockSpec((BLK, 1), lambda i: (i, 0)),
            pl.BlockSpec((BLK, 1), lambda i: (i + GRID, 0)),
            pl.BlockSpec((BLK, D), lambda i: (i, 0)),
            pl.BlockSpec((D, H), lambda i: (0, 0)),
        ],
        out_specs=pl.BlockSpec((BLK, H), lambda i: (i, 0)),
        out_shape=jax.ShapeDtypeStruct((NP, H), jnp.float32),
        compiler_params=pltpu.CompilerParams(
            dimension_semantics=("arbitrary",)),
    )(deg2, deg2, x_pad, W1)


def _tc2_body(degA, degB, aggA, aggB, y_ref, b_ref, g_ref, be_ref, rm_ref,
              rv_ref, w_ref, out_ref):
    dinv = lax.rsqrt(degA[...] + degB[...] + 1.0)
    t = dinv * (aggA[...] + aggB[...] + y_ref[...]) + b_ref[...]
    t = jnp.maximum(t, 0.0)
    t = (t - rm_ref[...]) * lax.rsqrt(rv_ref[...] + BN_EPS) * g_ref[...] \
        + be_ref[...]
    t = jnp.maximum(t, 0.0)
    h = jnp.dot(t, w_ref[...], preferred_element_type=jnp.float32)
    out_ref[...] = h * dinv


def _tc2(deg2, aggs, y1, b1, g1, be1, rm1, rv1, W2):
    row = pl.BlockSpec((1, H), lambda i: (0, 0))
    return pl.pallas_call(
        _tc2_body,
        grid=(GRID,),
        in_specs=[
            pl.BlockSpec((BLK, 1), lambda i: (i, 0)),
            pl.BlockSpec((BLK, 1), lambda i: (i + GRID, 0)),
            pl.BlockSpec((BLK, H), lambda i: (i, 0)),
            pl.BlockSpec((BLK, H), lambda i: (i + GRID, 0)),
            pl.BlockSpec((BLK, H), lambda i: (i, 0)),
            row, row, row, row, row,
            pl.BlockSpec((H, H), lambda i: (0, 0)),
        ],
        out_specs=pl.BlockSpec((BLK, H), lambda i: (i, 0)),
        out_shape=jax.ShapeDtypeStruct((NP, H), jnp.float32),
        compiler_params=pltpu.CompilerParams(
            dimension_semantics=("arbitrary",)),
    )(deg2, deg2, aggs, aggs, y1, b1, g1, be1, rm1, rv1, W2)


def _tc3_body(degA, degB, aggA, aggB, y_ref, b_ref, g_ref, be_ref, rm_ref,
              rv_ref, w_ref, b3_ref, logp_ref, var_ref, acc):
    i = pl.program_id(0)

    @pl.when(i == 0)
    def _():
        acc[0] = 0.0
        acc[1] = 0.0

    dinv = lax.rsqrt(degA[...] + degB[...] + 1.0)
    t = dinv * (aggA[...] + aggB[...] + y_ref[...]) + b_ref[...]
    t = jnp.maximum(t, 0.0)
    t = (t - rm_ref[...]) * lax.rsqrt(rv_ref[...] + BN_EPS) * g_ref[...] \
        + be_ref[...]
    t = jnp.maximum(t, 0.0)
    o = jnp.dot(t, w_ref[...], preferred_element_type=jnp.float32) + b3_ref[...]

    col = lax.broadcasted_iota(jnp.int32, (BLK, H), 1)
    colmask = col < C
    om = jnp.where(colmask, o, -1e30)
    m = jnp.max(om, axis=1, keepdims=True)
    ex = jnp.where(colmask, jnp.exp(o - m), 0.0)
    se = jnp.sum(ex, axis=1, keepdims=True)
    logp_ref[...] = o - m - jnp.log(se)

    rowg = lax.broadcasted_iota(jnp.int32, (BLK, H), 0) + i * BLK
    vmask = colmask & (rowg < N)
    ov = jnp.where(vmask, o, 0.0)
    acc[0] += jnp.sum(ov)
    acc[1] += jnp.sum(ov * ov)

    n = float(N * C)
    v = (acc[1] - acc[0] * acc[0] / n) / (n - 1.0)
    var_ref[...] = jnp.full((1, 1), v, jnp.float32)


def _tc3(deg2, aggs, y2, b2, g2, be2, rm2, rv2, W3p, b3p):
    row = pl.BlockSpec((1, H), lambda i: (0, 0))
    return pl.pallas_call(
        _tc3_body,
        grid=(GRID,),
        in_specs=[
            pl.BlockSpec((BLK, 1), lambda i: (i, 0)),
            pl.BlockSpec((BLK, 1), lambda i: (i + GRID, 0)),
            pl.BlockSpec((BLK, H), lambda i: (i, 0)),
            pl.BlockSpec((BLK, H), lambda i: (i + GRID, 0)),
            pl.BlockSpec((BLK, H), lambda i: (i, 0)),
            row, row, row, row, row,
            pl.BlockSpec((H, H), lambda i: (0, 0)),
            row,
        ],
        out_specs=[
            pl.BlockSpec((BLK, H), lambda i: (i, 0)),
            pl.BlockSpec((1, 1), lambda i: (0, 0)),
        ],
        out_shape=[
            jax.ShapeDtypeStruct((NP, H), jnp.float32),
            jax.ShapeDtypeStruct((1, 1), jnp.float32),
        ],
        scratch_shapes=[pltpu.SMEM((2,), jnp.float32)],
        compiler_params=pltpu.CompilerParams(
            dimension_semantics=("arbitrary",)),
    )(deg2, deg2, aggs, aggs, y2, b2, g2, be2, rm2, rv2, W3p, b3p)


def kernel(x, edge_index, W1, b1, g1, be1, rm1, rv1, W2, b2, g2, be2, rm2,
           rv2, W3, b3):
    src = edge_index[0]
    dst = edge_index[1]
    x_pad = jnp.pad(x, ((0, NP - N), (0, 0)))

    row = lambda v: v.reshape(1, H)
    W3p = jnp.pad(W3, ((0, 0), (0, H - C)))
    b3p = jnp.pad(b3, (0, H - C)).reshape(1, H)

    degs = _sc_degree(dst)
    deg2 = degs.reshape(NC * NP, 1)
    y1 = _tc1(deg2, x_pad, W1)
    aggs1 = _sc_aggregate(y1, src, dst)
    y2 = _tc2(deg2, aggs1, y1, row(b1), row(g1), row(be1), row(rm1),
              row(rv1), W2)
    aggs2 = _sc_aggregate(y2, src, dst)
    logp_pad, var2 = _tc3(deg2, aggs2, y2, row(b2), row(g2), row(be2),
                          row(rm2), row(rv2), W3p, b3p)
    return (logp_pad[:N, :C], var2[0, 0])
